# baseline (speedup 1.0000x reference)
import jax
import jax.numpy as jnp
from jax.experimental import pallas as pl
from jax.experimental.pallas import tpu as pltpu

_W = 128


def _body(x_hbm, o_ref, slab, fb, idx_smem, insem, fbsem, outsem):
    b, t, g, w = x_hbm.shape

    nd = 4
    slab_copies = []
    for k in range(b // nd):
        c = pltpu.make_async_copy(
            x_hbm.at[pl.ds(k * nd, nd), :, 0, :],
            slab.at[pl.ds(k * nd, nd)],
            insem.at[k],
        )
        c.start()
        slab_copies.append(c)

    waited = [False] * len(slab_copies)
    row_copies = []
    for i in range(b):
        if not waited[i // nd]:
            slab_copies[i // nd].wait()
            waited[i // nd] = True
        x = slab[i]
        m = jnp.max(jnp.abs(x), axis=1, keepdims=True)
        cnt = jnp.sum((m > 0.0).astype(jnp.int32))
        idx_smem[i] = jnp.maximum(cnt - 1, 0)

        @pl.when(cnt < t)
        def _():
            fc = pltpu.make_async_copy(x_hbm.at[i], fb, fbsem)
            fc.start()
            fc.wait()
            mf = jnp.max(jnp.abs(fb[...]), axis=(1, 2), keepdims=True)
            cf = jnp.sum((mf > 0.0).astype(jnp.int32))
            idx_smem[i] = jnp.maximum(cf - 1, 0)

        rc = pltpu.make_async_copy(
            x_hbm.at[i, pl.ds(idx_smem[i], 1)],
            o_ref.at[i],
            outsem.at[i],
        )
        rc.start()
        row_copies.append(rc)

    for c in row_copies:
        c.wait()


def kernel(inputs):
    b, t, f = inputs.shape
    g = f // _W
    x4 = inputs.reshape(b, t, g, _W)

    out = pl.pallas_call(
        _body,
        in_specs=[pl.BlockSpec(memory_space=pl.ANY)],
        out_specs=pl.BlockSpec((b, 1, g, _W), lambda: (0, 0, 0, 0)),
        out_shape=jax.ShapeDtypeStruct((b, 1, g, _W), jnp.float32),
        scratch_shapes=[
            pltpu.VMEM((b, t, _W), jnp.float32),
            pltpu.VMEM((t, g, _W), jnp.float32),
            pltpu.SMEM((b,), jnp.int32),
            pltpu.SemaphoreType.DMA((b,)),
            pltpu.SemaphoreType.DMA,
            pltpu.SemaphoreType.DMA((b,)),
        ],
    )(x4)

    return out.reshape(b, f)


# speedup vs baseline: 13.6612x; 13.6612x over previous
import jax
import jax.numpy as jnp
from jax.experimental import pallas as pl
from jax.experimental.pallas import tpu as pltpu

_W = 128
_ND = 4


def _body(x_hbm, o_ref, slab, fb, idx_smem, insem, fbsem, outsem):
    b, t, f = x_hbm.shape

    slab_copies = []
    for k in range(b // _ND):
        c = pltpu.make_async_copy(
            x_hbm.at[pl.ds(k * _ND, _ND), :, pl.ds(0, _W)],
            slab.at[pl.ds(k * _ND, _ND)],
            insem.at[k],
        )
        c.start()
        slab_copies.append(c)

    waited = [False] * len(slab_copies)
    row_copies = []
    for i in range(b):
        if not waited[i // _ND]:
            slab_copies[i // _ND].wait()
            waited[i // _ND] = True
        x = slab[i]
        m = jnp.max(jnp.abs(x), axis=1, keepdims=True)
        cnt = jnp.sum((m > 0.0).astype(jnp.int32))
        idx_smem[i] = jnp.maximum(cnt - 1, 0)

        @pl.when(cnt < t)
        def _():
            fc = pltpu.make_async_copy(x_hbm.at[i], fb, fbsem)
            fc.start()
            fc.wait()
            mf = jnp.max(jnp.abs(fb[...]), axis=1, keepdims=True)
            cf = jnp.sum((mf > 0.0).astype(jnp.int32))
            idx_smem[i] = jnp.maximum(cf - 1, 0)

        rc = pltpu.make_async_copy(
            x_hbm.at[i, pl.ds(idx_smem[i], 1), :],
            o_ref.at[pl.ds(i, 1)],
            outsem.at[i],
        )
        rc.start()
        row_copies.append(rc)

    for c in row_copies:
        c.wait()


def kernel(inputs):
    b, t, f = inputs.shape

    return pl.pallas_call(
        _body,
        in_specs=[pl.BlockSpec(memory_space=pl.ANY)],
        out_specs=pl.BlockSpec((b, f), lambda: (0, 0)),
        out_shape=jax.ShapeDtypeStruct((b, f), jnp.float32),
        scratch_shapes=[
            pltpu.VMEM((b, t, _W), jnp.float32),
            pltpu.VMEM((t, f), jnp.float32),
            pltpu.SMEM((b,), jnp.int32),
            pltpu.SemaphoreType.DMA((b,)),
            pltpu.SemaphoreType.DMA,
            pltpu.SemaphoreType.DMA((b,)),
        ],
    )(inputs)


# speedup vs baseline: 15.2825x; 1.1187x over previous
import jax
import jax.numpy as jnp
from jax.experimental import pallas as pl
from jax.experimental.pallas import tpu as pltpu

_W = 128
_ND = 1


def _body(x_hbm, o_ref, slab, fb, idx_smem, insem, fbsem, outsem):
    b, t, f = x_hbm.shape

    slab_copies = []
    for k in range(b // _ND):
        c = pltpu.make_async_copy(
            x_hbm.at[pl.ds(k * _ND, _ND), :, pl.ds(0, _W)],
            slab.at[pl.ds(k * _ND, _ND)],
            insem.at[k],
        )
        c.start()
        slab_copies.append(c)

    waited = [False] * len(slab_copies)
    row_copies = []
    for i in range(b):
        if not waited[i // _ND]:
            slab_copies[i // _ND].wait()
            waited[i // _ND] = True
        x = slab[i]
        m = jnp.max(jnp.abs(x), axis=1, keepdims=True)
        cnt = jnp.sum((m > 0.0).astype(jnp.int32))
        idx_smem[i] = jnp.maximum(cnt - 1, 0)

        @pl.when(cnt < t)
        def _():
            fc = pltpu.make_async_copy(x_hbm.at[i], fb, fbsem)
            fc.start()
            fc.wait()
            mf = jnp.max(jnp.abs(fb[...]), axis=1, keepdims=True)
            cf = jnp.sum((mf > 0.0).astype(jnp.int32))
            idx_smem[i] = jnp.maximum(cf - 1, 0)

        rc = pltpu.make_async_copy(
            x_hbm.at[i, pl.ds(idx_smem[i], 1), :],
            o_ref.at[pl.ds(i, 1)],
            outsem.at[i],
        )
        rc.start()
        row_copies.append(rc)

    for c in row_copies:
        c.wait()


def kernel(inputs):
    b, t, f = inputs.shape

    return pl.pallas_call(
        _body,
        in_specs=[pl.BlockSpec(memory_space=pl.ANY)],
        out_specs=pl.BlockSpec((b, f), lambda: (0, 0)),
        out_shape=jax.ShapeDtypeStruct((b, f), jnp.float32),
        scratch_shapes=[
            pltpu.VMEM((b, t, _W), jnp.float32),
            pltpu.VMEM((t, f), jnp.float32),
            pltpu.SMEM((b,), jnp.int32),
            pltpu.SemaphoreType.DMA((b,)),
            pltpu.SemaphoreType.DMA,
            pltpu.SemaphoreType.DMA((b,)),
        ],
    )(inputs)
